# Initial kernel scaffold; baseline (speedup 1.0000x reference)
import jax
import jax.numpy as jnp
from jax.experimental import pallas as pl


def kernel(vertices, xyz, normal, d0v, d0x, d0n, w1v, b1v, d1v, w1x, b1x, d1x, w1n, b1n, d1n, w2v, b2v, d2v, w2x, b2x, d2x, w2n, b2n, d2n, w3v, b3v, d3v, w3x, b3x, d3x, w3n, b3n, d3n, w4, b4, d4, w5, b5, d5):
    raise NotImplementedError("write your pallas kernel here")


# speedup vs baseline: 4.9251x; 4.9251x over previous
import functools
import math

import jax
import jax.numpy as jnp
from jax.experimental import pallas as pl

SUP = 7
NEI = 10
BIG = 3.0e38


def _dist(rows, ptsT):
    inner = jnp.dot(rows.astype(jnp.bfloat16), ptsT.astype(jnp.bfloat16),
                    preferred_element_type=jnp.float32)
    sq_r = jnp.sum(rows * rows, axis=1, keepdims=True)
    sq_c = jnp.sum(ptsT * ptsT, axis=0, keepdims=True)
    return sq_r - 2.0 * inner + sq_c


def _argmin_cols(dist, lane, V):
    m = jnp.min(dist, axis=1, keepdims=True)
    return jnp.min(jnp.where(dist == m, lane, jnp.int32(V)), axis=1)


def _topk_kernel(ptsT_ref, rows_ref, o_ref, *, k, skip, V):
    ptsT = ptsT_ref[0]
    rows = rows_ref[0]
    R = rows.shape[0]
    dist = _dist(rows, ptsT)
    lane = jax.lax.broadcasted_iota(jnp.int32, (R, V), 1)
    for j in range(k + skip):
        amin = _argmin_cols(dist, lane, V)
        if j >= skip:
            o_ref[0, :, j - skip] = amin
        dist = jnp.where(lane == amin[:, None], BIG, dist)


def _knn(pts, k):
    B, V, D = pts.shape
    ptsT = jnp.transpose(pts, (0, 2, 1))
    return pl.pallas_call(
        functools.partial(_topk_kernel, k=k, skip=1, V=V),
        grid=(B,),
        in_specs=[
            pl.BlockSpec((1, D, V), lambda b: (b, 0, 0)),
            pl.BlockSpec((1, V, D), lambda b: (b, 0, 0)),
        ],
        out_specs=pl.BlockSpec((1, V, k), lambda b: (b, 0, 0)),
        out_shape=jax.ShapeDtypeStruct((B, V, k), jnp.int32),
    )(ptsT, pts)


def _norm_cols(d):
    n = jnp.maximum(jnp.sqrt(jnp.sum(d * d, axis=0, keepdims=True)), 1e-12)
    return d / n


def _onehot(col, lane):
    return (col[:, None] == lane).astype(jnp.bfloat16)


def _split3(x):
    t1 = x.astype(jnp.bfloat16)
    r1 = x - t1.astype(jnp.float32)
    t2 = r1.astype(jnp.bfloat16)
    r2 = r1 - t2.astype(jnp.float32)
    t3 = r2.astype(jnp.bfloat16)
    return (t1, t2, t3)


def _gather3(oh, parts):
    out = jnp.dot(oh, parts[0], preferred_element_type=jnp.float32)
    for t in parts[1:]:
        out = out + jnp.dot(oh, t, preferred_element_type=jnp.float32)
    return out


def _mmul(a, b):
    return jnp.dot(a.astype(jnp.bfloat16), b.astype(jnp.bfloat16),
                   preferred_element_type=jnp.float32)


def _nd_theta(onehot, pts_parts, pts, dirs):
    nb = _gather3(onehot, pts_parts)
    d = nb - pts
    n = jnp.maximum(jnp.sqrt(jnp.sum(d * d, axis=1, keepdims=True)), 1e-12)
    return jnp.maximum(_mmul(d / n, dirs), 0.0)


def _sup_sum(acc, outc):
    red = acc[:, :outc]
    for s in range(1, SUP):
        red = red + acc[:, s * outc:(s + 1) * outc]
    return red


def _surface_kernel(idx_ref, pts9_ref, dv_ref, dx_ref, dn_ref, o_ref, *, V, n, kc):
    idx = idx_ref[0]
    pts9 = pts9_ref[0]
    dirs = [_norm_cols(dv_ref[...]), _norm_cols(dx_ref[...]), _norm_cols(dn_ref[...])]
    lane = jax.lax.broadcasted_iota(jnp.int32, (V, V), 1)
    pts9_parts = _split3(pts9)
    accs = [None, None, None]
    for j in range(n):
        oh = _onehot(idx[:, j], lane)
        nb9 = _gather3(oh, pts9_parts)
        for br in range(3):
            pts = pts9[:, 3 * br:3 * br + 3]
            d = nb9[:, 3 * br:3 * br + 3] - pts
            nn = jnp.maximum(jnp.sqrt(jnp.sum(d * d, axis=1, keepdims=True)), 1e-12)
            th = jnp.maximum(_mmul(d / nn, dirs[br]), 0.0)
            accs[br] = th if j == 0 else jnp.maximum(accs[br], th)
    for br in range(3):
        o_ref[0, :, br * kc:(br + 1) * kc] = jnp.maximum(_sup_sum(accs[br], kc), 0.0)


def _conv_surface3(idx, pts9, d0v, d0x, d0n, kc):
    B, V, _ = pts9.shape
    n = idx.shape[2]
    return pl.pallas_call(
        functools.partial(_surface_kernel, V=V, n=n, kc=kc),
        grid=(B,),
        in_specs=[
            pl.BlockSpec((1, V, n), lambda b: (b, 0, 0)),
            pl.BlockSpec((1, V, 9), lambda b: (b, 0, 0)),
            pl.BlockSpec(d0v.shape, lambda b: (0, 0)),
            pl.BlockSpec(d0x.shape, lambda b: (0, 0)),
            pl.BlockSpec(d0n.shape, lambda b: (0, 0)),
        ],
        out_specs=pl.BlockSpec((1, V, 3 * kc), lambda b: (b, 0, 0)),
        out_shape=jax.ShapeDtypeStruct((B, V, 3 * kc), jnp.float32),
    )(idx, pts9, d0v, d0x, d0n)


def _conv_layer_kernel(idx_ref, pts_ref, fmap_ref, w_ref, b_ref, d_ref, o_ref, *, V, n, outc):
    idx = idx_ref[0]
    pts = pts_ref[0]
    dirs = _norm_cols(d_ref[...])
    fo = _mmul(fmap_ref[0], w_ref[...]) + b_ref[...]
    center = fo[:, :outc]
    sup_parts = _split3(fo[:, outc:])
    pts_parts = _split3(pts)
    lane = jax.lax.broadcasted_iota(jnp.int32, (V, V), 1)
    acc = None
    for j in range(n):
        oh = _onehot(idx[:, j], lane)
        th = _nd_theta(oh, pts_parts, pts, dirs)
        g = _gather3(oh, sup_parts)
        t = th * g
        acc = t if j == 0 else jnp.maximum(acc, t)
    o_ref[0] = center + _sup_sum(acc, outc)


def _conv_layer(idx, pts, fmap, w, b, d, outc):
    B, V, D = pts.shape
    n = idx.shape[2]
    inc = fmap.shape[2]
    b2 = b.reshape(1, -1)
    return pl.pallas_call(
        functools.partial(_conv_layer_kernel, V=V, n=n, outc=outc),
        grid=(B,),
        in_specs=[
            pl.BlockSpec((1, V, n), lambda bb: (bb, 0, 0)),
            pl.BlockSpec((1, V, D), lambda bb: (bb, 0, 0)),
            pl.BlockSpec((1, V, inc), lambda bb: (bb, 0, 0)),
            pl.BlockSpec(w.shape, lambda bb: (0, 0)),
            pl.BlockSpec(b2.shape, lambda bb: (0, 0)),
            pl.BlockSpec(d.shape, lambda bb: (0, 0)),
        ],
        out_specs=pl.BlockSpec((1, V, outc), lambda bb: (bb, 0, 0)),
        out_shape=jax.ShapeDtypeStruct((B, V, outc), jnp.float32),
    )(idx, pts, fmap, w, b2, d)


def _bn_relu_kernel(x_ref, o_ref):
    x = x_ref[...]
    m = jnp.mean(x, axis=0, keepdims=True)
    v = jnp.mean((x - m) * (x - m), axis=0, keepdims=True)
    o_ref[...] = jnp.maximum((x - m) / jnp.sqrt(v + 1e-5), 0.0)


def _bn_relu(x):
    B, V, C = x.shape
    x2 = x.reshape(B * V, C)
    y = pl.pallas_call(
        _bn_relu_kernel,
        in_specs=[pl.BlockSpec(x2.shape, lambda: (0, 0))],
        out_specs=pl.BlockSpec(x2.shape, lambda: (0, 0)),
        out_shape=jax.ShapeDtypeStruct(x2.shape, jnp.float32),
    )(x2)
    return y.reshape(B, V, C)


def _pool_kernel(ptsT_ref, rows_ref, fmap_ref, o_ref, *, V, R, nnum):
    ptsT = ptsT_ref[0]
    rows = rows_ref[0]
    fmap = fmap_ref[0]
    dist = _dist(rows, ptsT)
    fmap_parts = _split3(fmap)
    lane = jax.lax.broadcasted_iota(jnp.int32, (R, V), 1)
    acc = None
    for j in range(nnum + 1):
        amin = _argmin_cols(dist, lane, V)
        if j >= 1:
            oh = _onehot(amin, lane)
            g = _gather3(oh, fmap_parts)
            acc = g if j == 1 else jnp.maximum(acc, g)
        dist = jnp.where(lane == amin[:, None], BIG, dist)
    o_ref[0] = acc


def _pool(pts, fmap, rate=4, nnum=4):
    B, V, D = pts.shape
    C = fmap.shape[2]
    R = V // rate
    rows = pts[:, ::rate, :]
    ptsT = jnp.transpose(pts, (0, 2, 1))
    return pl.pallas_call(
        functools.partial(_pool_kernel, V=V, R=R, nnum=nnum),
        grid=(B,),
        in_specs=[
            pl.BlockSpec((1, D, V), lambda b: (b, 0, 0)),
            pl.BlockSpec((1, R, D), lambda b: (b, 0, 0)),
            pl.BlockSpec((1, V, C), lambda b: (b, 0, 0)),
        ],
        out_specs=pl.BlockSpec((1, R, C), lambda b: (b, 0, 0)),
        out_shape=jax.ShapeDtypeStruct((B, R, C), jnp.float32),
    )(ptsT, rows, fmap)


def _up_kernel(tgt_ref, srcT_ref, feat_ref, o_ref, *, R):
    tgt = tgt_ref[0]
    srcT = srcT_ref[0]
    T = tgt.shape[0]
    dist = _dist(tgt, srcT)
    lane = jax.lax.broadcasted_iota(jnp.int32, (T, R), 1)
    amin = _argmin_cols(dist, lane, R)
    oh = _onehot(amin, lane)
    o_ref[0] = _gather3(oh, _split3(feat_ref[0]))


def _upsample(tgt, src, feat):
    B, T, D = tgt.shape
    R, C = src.shape[1], feat.shape[2]
    srcT = jnp.transpose(src, (0, 2, 1))
    return pl.pallas_call(
        functools.partial(_up_kernel, R=R),
        grid=(B,),
        in_specs=[
            pl.BlockSpec((1, T, D), lambda b: (b, 0, 0)),
            pl.BlockSpec((1, D, R), lambda b: (b, 0, 0)),
            pl.BlockSpec((1, R, C), lambda b: (b, 0, 0)),
        ],
        out_specs=pl.BlockSpec((1, T, C), lambda b: (b, 0, 0)),
        out_shape=jax.ShapeDtypeStruct((B, T, C), jnp.float32),
    )(tgt, srcT, feat)


def kernel(vertices, xyz, normal, d0v, d0x, d0n, w1v, b1v, d1v, w1x, b1x, d1x,
           w1n, b1n, d1n, w2v, b2v, d2v, w2x, b2x, d2x, w2n, b2n, d2n,
           w3v, b3v, d3v, w3x, b3x, d3x, w3n, b3n, d3n, w4, b4, d4, w5, b5, d5):
    B, N, _ = vertices.shape
    pts9 = jnp.concatenate([vertices, xyz, normal], axis=2)

    idx = _knn(vertices, NEI)
    fm0 = _conv_surface3(idx, pts9, d0v, d0x, d0n, 128)
    fm0v, fm0x, fm0n = fm0[:, :, :128], fm0[:, :, 128:256], fm0[:, :, 256:]

    fm1v = _bn_relu(_conv_layer(idx, vertices, fm0v, w1v, b1v, d1v, 128))
    fm1x = _bn_relu(_conv_layer(idx, xyz, fm0x, w1x, b1x, d1x, 128))
    fm1n = _bn_relu(_conv_layer(idx, normal, fm0n, w1n, b1n, d1n, 128))
    feat_1 = jnp.concatenate([fm1v, fm1x, fm1n], axis=2)

    v_pool_1 = vertices[:, ::4, :]
    x_pool_1 = xyz[:, ::4, :]
    n_pool_1 = normal[:, ::4, :]
    pool_1 = pts9[:, ::4, :]
    fmp_v = _pool(vertices, fm1v)
    fmp_x = _pool(xyz, fm1x)
    fmp_n = _pool(normal, fm1n)

    k2 = min(NEI, v_pool_1.shape[1] // 8)
    idx2 = _knn(v_pool_1, k2)
    fm2v = _bn_relu(_conv_layer(idx2, v_pool_1, fmp_v, w2v, b2v, d2v, 256))
    fm2x = _bn_relu(_conv_layer(idx2, x_pool_1, fmp_x, w2x, b2x, d2x, 256))
    fm2n = _bn_relu(_conv_layer(idx2, n_pool_1, fmp_n, w2n, b2n, d2n, 256))
    fm3v = _bn_relu(_conv_layer(idx2, v_pool_1, fm2v, w3v, b3v, d3v, 256))
    fm3x = _bn_relu(_conv_layer(idx2, x_pool_1, fm2x, w3x, b3x, d3x, 256))
    fm3n = _bn_relu(_conv_layer(idx2, n_pool_1, fm2n, w3n, b3n, d3n, 256))
    feat_2 = jnp.concatenate([fm3v, fm3x, fm3n], axis=2)

    pool_2 = pool_1[:, ::4, :]
    fm_pool_2 = _pool(pool_1, feat_2)

    k3 = min(NEI, pool_2.shape[1] // 8)
    idx3 = _knn(pool_2, k3)
    fm4 = _conv_layer(idx3, pool_2, fm_pool_2, w4, b4, d4, 256)
    fm5 = _conv_layer(idx3, pool_2, fm4, w5, b5, d5, 512)

    feat_12 = jnp.concatenate([feat_1[:, :feat_2.shape[1], :], feat_2], axis=2)
    f12 = _upsample(vertices, pool_1[:, :, :3], feat_12)
    f5 = _upsample(vertices, pool_2[:, :, :3], fm5)
    return jnp.concatenate([f5, f12], axis=2)


# speedup vs baseline: 5.5896x; 1.1349x over previous
import functools
import math

import jax
import jax.numpy as jnp
from jax.experimental import pallas as pl

SUP = 7
NEI = 10
BIG = 3.0e38


def _dist(rows, ptsT):
    inner = jnp.dot(rows.astype(jnp.bfloat16), ptsT.astype(jnp.bfloat16),
                    preferred_element_type=jnp.float32)
    sq_r = jnp.sum(rows * rows, axis=1, keepdims=True)
    sq_c = jnp.sum(ptsT * ptsT, axis=0, keepdims=True)
    return sq_r - 2.0 * inner + sq_c


def _argmin_cols(dist, lane, V):
    m = jnp.min(dist, axis=1, keepdims=True)
    return jnp.min(jnp.where(dist == m, lane, jnp.int32(V)), axis=1)


def _topk_kernel(ptsT_ref, rows_ref, o_ref, *, k, skip, V):
    ptsT = ptsT_ref[0]
    rows = rows_ref[0]
    R = rows.shape[0]
    dist = _dist(rows, ptsT)
    lane = jax.lax.broadcasted_iota(jnp.int32, (R, V), 1)
    for j in range(k + skip):
        amin = _argmin_cols(dist, lane, V)
        if j >= skip:
            o_ref[0, :, j - skip] = amin
        dist = jnp.where(lane == amin[:, None], BIG, dist)


def _knn(pts, k):
    B, V, D = pts.shape
    ptsT = jnp.transpose(pts, (0, 2, 1))
    return pl.pallas_call(
        functools.partial(_topk_kernel, k=k, skip=1, V=V),
        grid=(B,),
        in_specs=[
            pl.BlockSpec((1, D, V), lambda b: (b, 0, 0)),
            pl.BlockSpec((1, V, D), lambda b: (b, 0, 0)),
        ],
        out_specs=pl.BlockSpec((1, V, k), lambda b: (b, 0, 0)),
        out_shape=jax.ShapeDtypeStruct((B, V, k), jnp.int32),
    )(ptsT, pts)


def _norm_cols(d):
    n = jnp.maximum(jnp.sqrt(jnp.sum(d * d, axis=0, keepdims=True)), 1e-12)
    return d / n


def _onehot(col, lane):
    return (col[:, None] == lane).astype(jnp.bfloat16)


def _split3(x):
    t1 = x.astype(jnp.bfloat16)
    r1 = x - t1.astype(jnp.float32)
    t2 = r1.astype(jnp.bfloat16)
    r2 = r1 - t2.astype(jnp.float32)
    t3 = r2.astype(jnp.bfloat16)
    return (t1, t2, t3)


def _gather3(oh, parts):
    out = jnp.dot(oh, parts[0], preferred_element_type=jnp.float32)
    for t in parts[1:]:
        out = out + jnp.dot(oh, t, preferred_element_type=jnp.float32)
    return out


def _mmul(a, b):
    return jnp.dot(a.astype(jnp.bfloat16), b.astype(jnp.bfloat16),
                   preferred_element_type=jnp.float32)


def _nd_theta(onehot, pts_parts, pts, dirs):
    nb = _gather3(onehot, pts_parts)
    d = nb - pts
    n = jnp.maximum(jnp.sqrt(jnp.sum(d * d, axis=1, keepdims=True)), 1e-12)
    return jnp.maximum(_mmul(d / n, dirs), 0.0)


def _sup_sum(acc, outc):
    red = acc[:, :outc]
    for s in range(1, SUP):
        red = red + acc[:, s * outc:(s + 1) * outc]
    return red


def _surface_kernel(idx_ref, pts9_ref, dv_ref, dx_ref, dn_ref, o_ref, *, V, n, kc):
    idx = idx_ref[0]
    pts9 = pts9_ref[0]
    dirs = [_norm_cols(dv_ref[...]), _norm_cols(dx_ref[...]), _norm_cols(dn_ref[...])]
    lane = jax.lax.broadcasted_iota(jnp.int32, (V, V), 1)
    pts9_parts = _split3(pts9)
    accs = [None, None, None]
    for j in range(n):
        oh = _onehot(idx[:, j], lane)
        nb9 = _gather3(oh, pts9_parts)
        for br in range(3):
            pts = pts9[:, 3 * br:3 * br + 3]
            d = nb9[:, 3 * br:3 * br + 3] - pts
            nn = jnp.maximum(jnp.sqrt(jnp.sum(d * d, axis=1, keepdims=True)), 1e-12)
            th = jnp.maximum(_mmul(d / nn, dirs[br]), 0.0)
            accs[br] = th if j == 0 else jnp.maximum(accs[br], th)
    for br in range(3):
        o_ref[0, :, br * kc:(br + 1) * kc] = jnp.maximum(_sup_sum(accs[br], kc), 0.0)


def _conv_surface3(idx, pts9, d0v, d0x, d0n, kc):
    B, V, _ = pts9.shape
    n = idx.shape[2]
    return pl.pallas_call(
        functools.partial(_surface_kernel, V=V, n=n, kc=kc),
        grid=(B,),
        in_specs=[
            pl.BlockSpec((1, V, n), lambda b: (b, 0, 0)),
            pl.BlockSpec((1, V, 9), lambda b: (b, 0, 0)),
            pl.BlockSpec(d0v.shape, lambda b: (0, 0)),
            pl.BlockSpec(d0x.shape, lambda b: (0, 0)),
            pl.BlockSpec(d0n.shape, lambda b: (0, 0)),
        ],
        out_specs=pl.BlockSpec((1, V, 3 * kc), lambda b: (b, 0, 0)),
        out_shape=jax.ShapeDtypeStruct((B, V, 3 * kc), jnp.float32),
    )(idx, pts9, d0v, d0x, d0n)


def _conv3_kernel(idx_ref, pts9_ref, fm_ref, wv_ref, bv_ref, dv_ref,
                  wx_ref, bx_ref, dx_ref, wn_ref, bn_ref, dn_ref, o_ref,
                  *, V, VB, n, inc, outc):
    g = pl.program_id(1)
    idx = idx_ref[0]
    pts9 = pts9_ref[0]
    pts9_blk = pts9_ref[0, pl.ds(g * VB, VB), :]
    wbd = [(wv_ref, bv_ref, dv_ref), (wx_ref, bx_ref, dx_ref),
           (wn_ref, bn_ref, dn_ref)]
    centers, sup_parts, dirs = [], [], []
    for br, (w_r, b_r, d_r) in enumerate(wbd):
        fm_br = fm_ref[0][:, br * inc:(br + 1) * inc]
        fm_blk = fm_ref[0, pl.ds(g * VB, VB), br * inc:(br + 1) * inc]
        centers.append(_mmul(fm_blk, w_r[...][:, :outc]) + b_r[...][:, :outc])
        sup_parts.append(_split3(_mmul(fm_br, w_r[...][:, outc:])
                                 + b_r[...][:, outc:]))
        dirs.append(_norm_cols(d_r[...]))
    pts9_parts = _split3(pts9)
    lane = jax.lax.broadcasted_iota(jnp.int32, (VB, V), 1)
    accs = [None, None, None]
    for j in range(n):
        oh = _onehot(idx[:, j], lane)
        nb9 = _gather3(oh, pts9_parts)
        for br in range(3):
            pts = pts9_blk[:, 3 * br:3 * br + 3]
            d = nb9[:, 3 * br:3 * br + 3] - pts
            nn = jnp.maximum(jnp.sqrt(jnp.sum(d * d, axis=1, keepdims=True)), 1e-12)
            th = jnp.maximum(_mmul(d / nn, dirs[br]), 0.0)
            t = th * _gather3(oh, sup_parts[br])
            accs[br] = t if j == 0 else jnp.maximum(accs[br], t)
    for br in range(3):
        o_ref[0, :, br * outc:(br + 1) * outc] = centers[br] + _sup_sum(accs[br], outc)


def _conv3(idx, pts9, fm3, wv, bv, dv, wx, bx, dx, wn, bn, dn, outc, blocks=1):
    B, V, _ = pts9.shape
    n = idx.shape[2]
    inc = fm3.shape[2] // 3
    VB = V // blocks
    bvr, bxr, bnr = bv.reshape(1, -1), bx.reshape(1, -1), bn.reshape(1, -1)
    full = lambda a: pl.BlockSpec(a.shape, lambda bb, gg: tuple(0 for _ in a.shape))
    return pl.pallas_call(
        functools.partial(_conv3_kernel, V=V, VB=VB, n=n, inc=inc, outc=outc),
        grid=(B, blocks),
        in_specs=[
            pl.BlockSpec((1, VB, n), lambda bb, gg: (bb, gg, 0)),
            pl.BlockSpec((1, V, 9), lambda bb, gg: (bb, 0, 0)),
            pl.BlockSpec((1, V, 3 * inc), lambda bb, gg: (bb, 0, 0)),
            full(wv), full(bvr), full(dv),
            full(wx), full(bxr), full(dx),
            full(wn), full(bnr), full(dn),
        ],
        out_specs=pl.BlockSpec((1, VB, 3 * outc), lambda bb, gg: (bb, gg, 0)),
        out_shape=jax.ShapeDtypeStruct((B, V, 3 * outc), jnp.float32),
    )(idx, pts9, fm3, wv, bvr, dv, wx, bxr, dx, wn, bnr, dn)


def _conv_layer_kernel(idx_ref, pts_ref, fmap_ref, w_ref, b_ref, d_ref, o_ref, *, V, n, outc):
    idx = idx_ref[0]
    pts = pts_ref[0]
    dirs = _norm_cols(d_ref[...])
    fo = _mmul(fmap_ref[0], w_ref[...]) + b_ref[...]
    center = fo[:, :outc]
    sup_parts = _split3(fo[:, outc:])
    pts_parts = _split3(pts)
    lane = jax.lax.broadcasted_iota(jnp.int32, (V, V), 1)
    acc = None
    for j in range(n):
        oh = _onehot(idx[:, j], lane)
        th = _nd_theta(oh, pts_parts, pts, dirs)
        g = _gather3(oh, sup_parts)
        t = th * g
        acc = t if j == 0 else jnp.maximum(acc, t)
    o_ref[0] = center + _sup_sum(acc, outc)


def _conv_layer(idx, pts, fmap, w, b, d, outc):
    B, V, D = pts.shape
    n = idx.shape[2]
    inc = fmap.shape[2]
    b2 = b.reshape(1, -1)
    return pl.pallas_call(
        functools.partial(_conv_layer_kernel, V=V, n=n, outc=outc),
        grid=(B,),
        in_specs=[
            pl.BlockSpec((1, V, n), lambda bb: (bb, 0, 0)),
            pl.BlockSpec((1, V, D), lambda bb: (bb, 0, 0)),
            pl.BlockSpec((1, V, inc), lambda bb: (bb, 0, 0)),
            pl.BlockSpec(w.shape, lambda bb: (0, 0)),
            pl.BlockSpec(b2.shape, lambda bb: (0, 0)),
            pl.BlockSpec(d.shape, lambda bb: (0, 0)),
        ],
        out_specs=pl.BlockSpec((1, V, outc), lambda bb: (bb, 0, 0)),
        out_shape=jax.ShapeDtypeStruct((B, V, outc), jnp.float32),
    )(idx, pts, fmap, w, b2, d)


def _bn_relu_kernel(x_ref, o_ref):
    x = x_ref[...]
    m = jnp.mean(x, axis=0, keepdims=True)
    v = jnp.mean((x - m) * (x - m), axis=0, keepdims=True)
    o_ref[...] = jnp.maximum((x - m) / jnp.sqrt(v + 1e-5), 0.0)


def _bn_relu(x):
    B, V, C = x.shape
    x2 = x.reshape(B * V, C)
    y = pl.pallas_call(
        _bn_relu_kernel,
        in_specs=[pl.BlockSpec(x2.shape, lambda: (0, 0))],
        out_specs=pl.BlockSpec(x2.shape, lambda: (0, 0)),
        out_shape=jax.ShapeDtypeStruct(x2.shape, jnp.float32),
    )(x2)
    return y.reshape(B, V, C)


def _pool_kernel(ptsT_ref, rows_ref, fmap_ref, o_ref, *, V, R, nnum):
    ptsT = ptsT_ref[0]
    rows = rows_ref[0]
    fmap = fmap_ref[0]
    dist = _dist(rows, ptsT)
    fmap_parts = _split3(fmap)
    lane = jax.lax.broadcasted_iota(jnp.int32, (R, V), 1)
    acc = None
    for j in range(nnum + 1):
        amin = _argmin_cols(dist, lane, V)
        if j >= 1:
            oh = _onehot(amin, lane)
            g = _gather3(oh, fmap_parts)
            acc = g if j == 1 else jnp.maximum(acc, g)
        dist = jnp.where(lane == amin[:, None], BIG, dist)
    o_ref[0] = acc


def _pool(pts, fmap, rate=4, nnum=4):
    B, V, D = pts.shape
    C = fmap.shape[2]
    R = V // rate
    rows = pts[:, ::rate, :]
    ptsT = jnp.transpose(pts, (0, 2, 1))
    return pl.pallas_call(
        functools.partial(_pool_kernel, V=V, R=R, nnum=nnum),
        grid=(B,),
        in_specs=[
            pl.BlockSpec((1, D, V), lambda b: (b, 0, 0)),
            pl.BlockSpec((1, R, D), lambda b: (b, 0, 0)),
            pl.BlockSpec((1, V, C), lambda b: (b, 0, 0)),
        ],
        out_specs=pl.BlockSpec((1, R, C), lambda b: (b, 0, 0)),
        out_shape=jax.ShapeDtypeStruct((B, R, C), jnp.float32),
    )(ptsT, rows, fmap)


def _up_kernel(tgt_ref, srcT_ref, feat_ref, o_ref, *, R):
    tgt = tgt_ref[0]
    srcT = srcT_ref[0]
    T = tgt.shape[0]
    dist = _dist(tgt, srcT)
    lane = jax.lax.broadcasted_iota(jnp.int32, (T, R), 1)
    amin = _argmin_cols(dist, lane, R)
    oh = _onehot(amin, lane)
    o_ref[0] = _gather3(oh, _split3(feat_ref[0]))


def _upsample(tgt, src, feat):
    B, T, D = tgt.shape
    R, C = src.shape[1], feat.shape[2]
    srcT = jnp.transpose(src, (0, 2, 1))
    return pl.pallas_call(
        functools.partial(_up_kernel, R=R),
        grid=(B,),
        in_specs=[
            pl.BlockSpec((1, T, D), lambda b: (b, 0, 0)),
            pl.BlockSpec((1, D, R), lambda b: (b, 0, 0)),
            pl.BlockSpec((1, R, C), lambda b: (b, 0, 0)),
        ],
        out_specs=pl.BlockSpec((1, T, C), lambda b: (b, 0, 0)),
        out_shape=jax.ShapeDtypeStruct((B, T, C), jnp.float32),
    )(tgt, srcT, feat)


def kernel(vertices, xyz, normal, d0v, d0x, d0n, w1v, b1v, d1v, w1x, b1x, d1x,
           w1n, b1n, d1n, w2v, b2v, d2v, w2x, b2x, d2x, w2n, b2n, d2n,
           w3v, b3v, d3v, w3x, b3x, d3x, w3n, b3n, d3n, w4, b4, d4, w5, b5, d5):
    B, N, _ = vertices.shape
    pts9 = jnp.concatenate([vertices, xyz, normal], axis=2)

    idx = _knn(vertices, NEI)
    fm0 = _conv_surface3(idx, pts9, d0v, d0x, d0n, 128)
    fm0v, fm0x, fm0n = fm0[:, :, :128], fm0[:, :, 128:256], fm0[:, :, 256:]

    feat_1 = _bn_relu(_conv3(idx, pts9, fm0, w1v, b1v, d1v, w1x, b1x, d1x,
                             w1n, b1n, d1n, 128, blocks=2))
    fm1v, fm1x, fm1n = (feat_1[:, :, :128], feat_1[:, :, 128:256],
                        feat_1[:, :, 256:])

    v_pool_1 = vertices[:, ::4, :]
    x_pool_1 = xyz[:, ::4, :]
    n_pool_1 = normal[:, ::4, :]
    pool_1 = pts9[:, ::4, :]
    fmp_v = _pool(vertices, fm1v)
    fmp_x = _pool(xyz, fm1x)
    fmp_n = _pool(normal, fm1n)

    k2 = min(NEI, v_pool_1.shape[1] // 8)
    idx2 = _knn(v_pool_1, k2)
    pts9p = pts9[:, ::4, :]
    fmp = jnp.concatenate([fmp_v, fmp_x, fmp_n], axis=2)
    fm2 = _bn_relu(_conv3(idx2, pts9p, fmp, w2v, b2v, d2v, w2x, b2x, d2x,
                          w2n, b2n, d2n, 256))
    feat_2 = _bn_relu(_conv3(idx2, pts9p, fm2, w3v, b3v, d3v, w3x, b3x, d3x,
                             w3n, b3n, d3n, 256))

    pool_2 = pool_1[:, ::4, :]
    fm_pool_2 = _pool(pool_1, feat_2)

    k3 = min(NEI, pool_2.shape[1] // 8)
    idx3 = _knn(pool_2, k3)
    fm4 = _conv_layer(idx3, pool_2, fm_pool_2, w4, b4, d4, 256)
    fm5 = _conv_layer(idx3, pool_2, fm4, w5, b5, d5, 512)

    feat_12 = jnp.concatenate([feat_1[:, :feat_2.shape[1], :], feat_2], axis=2)
    f12 = _upsample(vertices, pool_1[:, :, :3], feat_12)
    f5 = _upsample(vertices, pool_2[:, :, :3], fm5)
    return jnp.concatenate([f5, f12], axis=2)


# speedup vs baseline: 6.6012x; 1.1810x over previous
import functools
import math

import jax
import jax.numpy as jnp
from jax.experimental import pallas as pl

SUP = 7
NEI = 10
BIG = 3.0e38


def _dist(rows, ptsT):
    inner = jnp.dot(rows.astype(jnp.bfloat16), ptsT.astype(jnp.bfloat16),
                    preferred_element_type=jnp.float32)
    sq_r = jnp.sum(rows * rows, axis=1, keepdims=True)
    sq_c = jnp.sum(ptsT * ptsT, axis=0, keepdims=True)
    return sq_r - 2.0 * inner + sq_c


def _argmin_cols(dist, lane, V):
    m = jnp.min(dist, axis=1, keepdims=True)
    return jnp.min(jnp.where(dist == m, lane, jnp.int32(V)), axis=1)


def _topk_kernel(ptsT_ref, rows_ref, o_ref, *, k, skip, V):
    ptsT = ptsT_ref[0]
    rows = rows_ref[0]
    R = rows.shape[0]
    dist = _dist(rows, ptsT)
    lane = jax.lax.broadcasted_iota(jnp.int32, (R, V), 1)
    for j in range(k + skip):
        amin = _argmin_cols(dist, lane, V)
        if j >= skip:
            o_ref[0, :, j - skip] = amin
        dist = jnp.where(lane == amin[:, None], BIG, dist)


def _knn(pts, k):
    B, V, D = pts.shape
    ptsT = jnp.transpose(pts, (0, 2, 1))
    return pl.pallas_call(
        functools.partial(_topk_kernel, k=k, skip=1, V=V),
        grid=(B,),
        in_specs=[
            pl.BlockSpec((1, D, V), lambda b: (b, 0, 0)),
            pl.BlockSpec((1, V, D), lambda b: (b, 0, 0)),
        ],
        out_specs=pl.BlockSpec((1, V, k), lambda b: (b, 0, 0)),
        out_shape=jax.ShapeDtypeStruct((B, V, k), jnp.int32),
    )(ptsT, pts)


def _norm_cols(d):
    n = jnp.maximum(jnp.sqrt(jnp.sum(d * d, axis=0, keepdims=True)), 1e-12)
    return d / n


def _onehot(col, lane):
    return (col[:, None] == lane).astype(jnp.bfloat16)


def _split3(x):
    t1 = x.astype(jnp.bfloat16)
    r1 = x - t1.astype(jnp.float32)
    t2 = r1.astype(jnp.bfloat16)
    r2 = r1 - t2.astype(jnp.float32)
    t3 = r2.astype(jnp.bfloat16)
    return (t1, t2, t3)


def _split2(x):
    t1 = x.astype(jnp.bfloat16)
    r1 = x - t1.astype(jnp.float32)
    return (t1, r1.astype(jnp.bfloat16))


def _gather3(oh, parts):
    out = jnp.dot(oh, parts[0], preferred_element_type=jnp.float32)
    for t in parts[1:]:
        out = out + jnp.dot(oh, t, preferred_element_type=jnp.float32)
    return out


def _mmul(a, b):
    return jnp.dot(a.astype(jnp.bfloat16), b.astype(jnp.bfloat16),
                   preferred_element_type=jnp.float32)


def _nd_theta(onehot, pts_parts, pts, dirs):
    nb = _gather3(onehot, pts_parts)
    d = nb - pts
    n = jnp.maximum(jnp.sqrt(jnp.sum(d * d, axis=1, keepdims=True)), 1e-12)
    return jnp.maximum(_mmul(d / n, dirs), 0.0)


def _sup_sum(acc, outc):
    red = acc[:, :outc]
    for s in range(1, SUP):
        red = red + acc[:, s * outc:(s + 1) * outc]
    return red


def _surface_kernel(idx_ref, pts9_ref, dv_ref, dx_ref, dn_ref, o_ref, *, V, n, kc):
    idx = idx_ref[0]
    pts9 = pts9_ref[0]
    dirs = [_norm_cols(dv_ref[...]), _norm_cols(dx_ref[...]), _norm_cols(dn_ref[...])]
    lane = jax.lax.broadcasted_iota(jnp.int32, (V, V), 1)
    pts9_parts = _split3(pts9)
    accs = [None, None, None]
    for j in range(n):
        oh = _onehot(idx[:, j], lane)
        nb9 = _gather3(oh, pts9_parts)
        for br in range(3):
            pts = pts9[:, 3 * br:3 * br + 3]
            d = nb9[:, 3 * br:3 * br + 3] - pts
            nn = jnp.maximum(jnp.sqrt(jnp.sum(d * d, axis=1, keepdims=True)), 1e-12)
            th = jnp.maximum(_mmul(d / nn, dirs[br]), 0.0)
            accs[br] = th if j == 0 else jnp.maximum(accs[br], th)
    for br in range(3):
        o_ref[0, :, br * kc:(br + 1) * kc] = jnp.maximum(_sup_sum(accs[br], kc), 0.0)


def _conv_surface3(idx, pts9, d0v, d0x, d0n, kc):
    B, V, _ = pts9.shape
    n = idx.shape[2]
    return pl.pallas_call(
        functools.partial(_surface_kernel, V=V, n=n, kc=kc),
        grid=(B,),
        in_specs=[
            pl.BlockSpec((1, V, n), lambda b: (b, 0, 0)),
            pl.BlockSpec((1, V, 9), lambda b: (b, 0, 0)),
            pl.BlockSpec(d0v.shape, lambda b: (0, 0)),
            pl.BlockSpec(d0x.shape, lambda b: (0, 0)),
            pl.BlockSpec(d0n.shape, lambda b: (0, 0)),
        ],
        out_specs=pl.BlockSpec((1, V, 3 * kc), lambda b: (b, 0, 0)),
        out_shape=jax.ShapeDtypeStruct((B, V, 3 * kc), jnp.float32),
    )(idx, pts9, d0v, d0x, d0n)


def _conv3_kernel(idx_ref, pts9_ref, fm_ref, wv_ref, bv_ref, dv_ref,
                  wx_ref, bx_ref, dx_ref, wn_ref, bn_ref, dn_ref, o_ref,
                  *, V, VB, n, inc, outc):
    g = pl.program_id(1)
    idx = idx_ref[0]
    pts9 = pts9_ref[0]
    pts9_blk = pts9_ref[0, pl.ds(g * VB, VB), :]
    wbd = [(wv_ref, bv_ref, dv_ref), (wx_ref, bx_ref, dx_ref),
           (wn_ref, bn_ref, dn_ref)]
    centers, sup_parts, dirs = [], [], []
    for br, (w_r, b_r, d_r) in enumerate(wbd):
        fm_br = fm_ref[0][:, br * inc:(br + 1) * inc]
        fm_blk = fm_ref[0, pl.ds(g * VB, VB), br * inc:(br + 1) * inc]
        centers.append(_mmul(fm_blk, w_r[...][:, :outc]) + b_r[...][:, :outc])
        sup_parts.append(_split2(_mmul(fm_br, w_r[...][:, outc:])
                                 + b_r[...][:, outc:]))
        dirs.append(_norm_cols(d_r[...]))
    pts9_parts = _split3(pts9)
    lane = jax.lax.broadcasted_iota(jnp.int32, (VB, V), 1)
    accs = [None, None, None]
    for j in range(n):
        oh = _onehot(idx[:, j], lane)
        nb9 = _gather3(oh, pts9_parts)
        for br in range(3):
            pts = pts9_blk[:, 3 * br:3 * br + 3]
            d = nb9[:, 3 * br:3 * br + 3] - pts
            nn = jnp.maximum(jnp.sqrt(jnp.sum(d * d, axis=1, keepdims=True)), 1e-12)
            th = jnp.maximum(_mmul(d / nn, dirs[br]), 0.0)
            t = th * _gather3(oh, sup_parts[br])
            accs[br] = t if j == 0 else jnp.maximum(accs[br], t)
    for br in range(3):
        o_ref[0, :, br * outc:(br + 1) * outc] = centers[br] + _sup_sum(accs[br], outc)


def _conv3(idx, pts9, fm3, wv, bv, dv, wx, bx, dx, wn, bn, dn, outc, blocks=1):
    B, V, _ = pts9.shape
    n = idx.shape[2]
    inc = fm3.shape[2] // 3
    VB = V // blocks
    bvr, bxr, bnr = bv.reshape(1, -1), bx.reshape(1, -1), bn.reshape(1, -1)
    full = lambda a: pl.BlockSpec(a.shape, lambda bb, gg: tuple(0 for _ in a.shape))
    return pl.pallas_call(
        functools.partial(_conv3_kernel, V=V, VB=VB, n=n, inc=inc, outc=outc),
        grid=(B, blocks),
        in_specs=[
            pl.BlockSpec((1, VB, n), lambda bb, gg: (bb, gg, 0)),
            pl.BlockSpec((1, V, 9), lambda bb, gg: (bb, 0, 0)),
            pl.BlockSpec((1, V, 3 * inc), lambda bb, gg: (bb, 0, 0)),
            full(wv), full(bvr), full(dv),
            full(wx), full(bxr), full(dx),
            full(wn), full(bnr), full(dn),
        ],
        out_specs=pl.BlockSpec((1, VB, 3 * outc), lambda bb, gg: (bb, gg, 0)),
        out_shape=jax.ShapeDtypeStruct((B, V, 3 * outc), jnp.float32),
    )(idx, pts9, fm3, wv, bvr, dv, wx, bxr, dx, wn, bnr, dn)


def _conv_layer_kernel(idx_ref, pts_ref, fmap_ref, w_ref, b_ref, d_ref, o_ref, *, V, n, outc):
    idx = idx_ref[0]
    pts = pts_ref[0]
    dirs = _norm_cols(d_ref[...])
    fo = _mmul(fmap_ref[0], w_ref[...]) + b_ref[...]
    center = fo[:, :outc]
    sup_parts = _split3(fo[:, outc:])
    pts_parts = _split3(pts)
    lane = jax.lax.broadcasted_iota(jnp.int32, (V, V), 1)
    acc = None
    for j in range(n):
        oh = _onehot(idx[:, j], lane)
        th = _nd_theta(oh, pts_parts, pts, dirs)
        g = _gather3(oh, sup_parts)
        t = th * g
        acc = t if j == 0 else jnp.maximum(acc, t)
    o_ref[0] = center + _sup_sum(acc, outc)


def _conv_layer(idx, pts, fmap, w, b, d, outc):
    B, V, D = pts.shape
    n = idx.shape[2]
    inc = fmap.shape[2]
    b2 = b.reshape(1, -1)
    return pl.pallas_call(
        functools.partial(_conv_layer_kernel, V=V, n=n, outc=outc),
        grid=(B,),
        in_specs=[
            pl.BlockSpec((1, V, n), lambda bb: (bb, 0, 0)),
            pl.BlockSpec((1, V, D), lambda bb: (bb, 0, 0)),
            pl.BlockSpec((1, V, inc), lambda bb: (bb, 0, 0)),
            pl.BlockSpec(w.shape, lambda bb: (0, 0)),
            pl.BlockSpec(b2.shape, lambda bb: (0, 0)),
            pl.BlockSpec(d.shape, lambda bb: (0, 0)),
        ],
        out_specs=pl.BlockSpec((1, V, outc), lambda bb: (bb, 0, 0)),
        out_shape=jax.ShapeDtypeStruct((B, V, outc), jnp.float32),
    )(idx, pts, fmap, w, b2, d)


def _bn_relu_kernel(x_ref, o_ref):
    x = x_ref[...]
    m = jnp.mean(x, axis=0, keepdims=True)
    v = jnp.mean((x - m) * (x - m), axis=0, keepdims=True)
    o_ref[...] = jnp.maximum((x - m) / jnp.sqrt(v + 1e-5), 0.0)


def _bn_relu(x):
    B, V, C = x.shape
    x2 = x.reshape(B * V, C)
    y = pl.pallas_call(
        _bn_relu_kernel,
        in_specs=[pl.BlockSpec(x2.shape, lambda: (0, 0))],
        out_specs=pl.BlockSpec(x2.shape, lambda: (0, 0)),
        out_shape=jax.ShapeDtypeStruct(x2.shape, jnp.float32),
    )(x2)
    return y.reshape(B, V, C)


def _pool_kernel(ptsT_ref, rows_ref, fmap_ref, o_ref, *, V, R, nnum):
    ptsT = ptsT_ref[0]
    rows = rows_ref[0]
    fmap = fmap_ref[0]
    dist = _dist(rows, ptsT)
    fmap_parts = _split2(fmap)
    lane = jax.lax.broadcasted_iota(jnp.int32, (R, V), 1)
    acc = None
    for j in range(nnum + 1):
        amin = _argmin_cols(dist, lane, V)
        if j >= 1:
            oh = _onehot(amin, lane)
            g = _gather3(oh, fmap_parts)
            acc = g if j == 1 else jnp.maximum(acc, g)
        dist = jnp.where(lane == amin[:, None], BIG, dist)
    o_ref[0] = acc


def _pool(pts, fmap, rate=4, nnum=4):
    B, V, D = pts.shape
    C = fmap.shape[2]
    R = V // rate
    rows = pts[:, ::rate, :]
    ptsT = jnp.transpose(pts, (0, 2, 1))
    return pl.pallas_call(
        functools.partial(_pool_kernel, V=V, R=R, nnum=nnum),
        grid=(B,),
        in_specs=[
            pl.BlockSpec((1, D, V), lambda b: (b, 0, 0)),
            pl.BlockSpec((1, R, D), lambda b: (b, 0, 0)),
            pl.BlockSpec((1, V, C), lambda b: (b, 0, 0)),
        ],
        out_specs=pl.BlockSpec((1, R, C), lambda b: (b, 0, 0)),
        out_shape=jax.ShapeDtypeStruct((B, R, C), jnp.float32),
    )(ptsT, rows, fmap)


def _up_kernel(tgt_ref, srcT_ref, feat_ref, o_ref, *, R):
    tgt = tgt_ref[0]
    srcT = srcT_ref[0]
    T = tgt.shape[0]
    dist = _dist(tgt, srcT)
    lane = jax.lax.broadcasted_iota(jnp.int32, (T, R), 1)
    amin = _argmin_cols(dist, lane, R)
    oh = _onehot(amin, lane)
    o_ref[0] = _gather3(oh, _split2(feat_ref[0]))


def _upsample(tgt, src, feat):
    B, T, D = tgt.shape
    R, C = src.shape[1], feat.shape[2]
    srcT = jnp.transpose(src, (0, 2, 1))
    return pl.pallas_call(
        functools.partial(_up_kernel, R=R),
        grid=(B,),
        in_specs=[
            pl.BlockSpec((1, T, D), lambda b: (b, 0, 0)),
            pl.BlockSpec((1, D, R), lambda b: (b, 0, 0)),
            pl.BlockSpec((1, R, C), lambda b: (b, 0, 0)),
        ],
        out_specs=pl.BlockSpec((1, T, C), lambda b: (b, 0, 0)),
        out_shape=jax.ShapeDtypeStruct((B, T, C), jnp.float32),
    )(tgt, srcT, feat)


def kernel(vertices, xyz, normal, d0v, d0x, d0n, w1v, b1v, d1v, w1x, b1x, d1x,
           w1n, b1n, d1n, w2v, b2v, d2v, w2x, b2x, d2x, w2n, b2n, d2n,
           w3v, b3v, d3v, w3x, b3x, d3x, w3n, b3n, d3n, w4, b4, d4, w5, b5, d5):
    B, N, _ = vertices.shape
    pts9 = jnp.concatenate([vertices, xyz, normal], axis=2)

    idx = _knn(vertices, NEI)
    fm0 = _conv_surface3(idx, pts9, d0v, d0x, d0n, 128)
    fm0v, fm0x, fm0n = fm0[:, :, :128], fm0[:, :, 128:256], fm0[:, :, 256:]

    feat_1 = _bn_relu(_conv3(idx, pts9, fm0, w1v, b1v, d1v, w1x, b1x, d1x,
                             w1n, b1n, d1n, 128, blocks=2))
    fm1v, fm1x, fm1n = (feat_1[:, :, :128], feat_1[:, :, 128:256],
                        feat_1[:, :, 256:])

    v_pool_1 = vertices[:, ::4, :]
    x_pool_1 = xyz[:, ::4, :]
    n_pool_1 = normal[:, ::4, :]
    pool_1 = pts9[:, ::4, :]
    fmp_v = _pool(vertices, fm1v)
    fmp_x = _pool(xyz, fm1x)
    fmp_n = _pool(normal, fm1n)

    k2 = min(NEI, v_pool_1.shape[1] // 8)
    idx2 = _knn(v_pool_1, k2)
    pts9p = pts9[:, ::4, :]
    fmp = jnp.concatenate([fmp_v, fmp_x, fmp_n], axis=2)
    fm2 = _bn_relu(_conv3(idx2, pts9p, fmp, w2v, b2v, d2v, w2x, b2x, d2x,
                          w2n, b2n, d2n, 256))
    feat_2 = _bn_relu(_conv3(idx2, pts9p, fm2, w3v, b3v, d3v, w3x, b3x, d3x,
                             w3n, b3n, d3n, 256))

    pool_2 = pool_1[:, ::4, :]
    fm_pool_2 = _pool(pool_1, feat_2)

    k3 = min(NEI, pool_2.shape[1] // 8)
    idx3 = _knn(pool_2, k3)
    fm4 = _conv_layer(idx3, pool_2, fm_pool_2, w4, b4, d4, 256)
    fm5 = _conv_layer(idx3, pool_2, fm4, w5, b5, d5, 512)

    feat_12 = jnp.concatenate([feat_1[:, :feat_2.shape[1], :], feat_2], axis=2)
    f12 = _upsample(vertices, pool_1[:, :, :3], feat_12)
    f5 = _upsample(vertices, pool_2[:, :, :3], fm5)
    return jnp.concatenate([f5, f12], axis=2)


# speedup vs baseline: 6.6222x; 1.0032x over previous
import functools
import math

import jax
import jax.numpy as jnp
from jax.experimental import pallas as pl

SUP = 7
NEI = 10
BIG = 3.0e38


def _dist(rows, ptsT):
    inner = jnp.dot(rows.astype(jnp.bfloat16), ptsT.astype(jnp.bfloat16),
                    preferred_element_type=jnp.float32)
    sq_r = jnp.sum(rows * rows, axis=1, keepdims=True)
    sq_c = jnp.sum(ptsT * ptsT, axis=0, keepdims=True)
    return sq_r - 2.0 * inner + sq_c


def _argmin_cols(dist, lane, V):
    m = jnp.min(dist, axis=1, keepdims=True)
    return jnp.min(jnp.where(dist == m, lane, jnp.int32(V)), axis=1)


def _topk_kernel(ptsT_ref, rows_ref, o_ref, *, k, skip, V):
    ptsT = ptsT_ref[0]
    rows = rows_ref[0]
    R = rows.shape[0]
    dist = _dist(rows, ptsT)
    lane = jax.lax.broadcasted_iota(jnp.int32, (R, V), 1)
    for j in range(k + skip):
        amin = _argmin_cols(dist, lane, V)
        if j >= skip:
            o_ref[0, :, j - skip] = amin
        dist = jnp.where(lane == amin[:, None], BIG, dist)


def _knn(pts, k):
    B, V, D = pts.shape
    ptsT = jnp.transpose(pts, (0, 2, 1))
    return pl.pallas_call(
        functools.partial(_topk_kernel, k=k, skip=1, V=V),
        grid=(B,),
        in_specs=[
            pl.BlockSpec((1, D, V), lambda b: (b, 0, 0)),
            pl.BlockSpec((1, V, D), lambda b: (b, 0, 0)),
        ],
        out_specs=pl.BlockSpec((1, V, k), lambda b: (b, 0, 0)),
        out_shape=jax.ShapeDtypeStruct((B, V, k), jnp.int32),
    )(ptsT, pts)


def _norm_cols(d):
    n = jnp.maximum(jnp.sqrt(jnp.sum(d * d, axis=0, keepdims=True)), 1e-12)
    return d / n


def _onehot(col, lane):
    return (col[:, None] == lane).astype(jnp.bfloat16)


def _split3(x):
    t1 = x.astype(jnp.bfloat16)
    r1 = x - t1.astype(jnp.float32)
    t2 = r1.astype(jnp.bfloat16)
    r2 = r1 - t2.astype(jnp.float32)
    t3 = r2.astype(jnp.bfloat16)
    return (t1, t2, t3)


def _split2(x):
    t1 = x.astype(jnp.bfloat16)
    r1 = x - t1.astype(jnp.float32)
    return (t1, r1.astype(jnp.bfloat16))


def _gather3(oh, parts):
    out = jnp.dot(oh, parts[0], preferred_element_type=jnp.float32)
    for t in parts[1:]:
        out = out + jnp.dot(oh, t, preferred_element_type=jnp.float32)
    return out


def _mmul(a, b):
    return jnp.dot(a.astype(jnp.bfloat16), b.astype(jnp.bfloat16),
                   preferred_element_type=jnp.float32)


def _nd_theta(onehot, pts_parts, pts, dirs):
    nb = _gather3(onehot, pts_parts)
    d = nb - pts
    n = jnp.maximum(jnp.sqrt(jnp.sum(d * d, axis=1, keepdims=True)), 1e-12)
    return jnp.maximum(_mmul(d / n, dirs), 0.0)


def _sup_sum(acc, outc):
    red = acc[:, :outc]
    for s in range(1, SUP):
        red = red + acc[:, s * outc:(s + 1) * outc]
    return red


def _surface_kernel(idx_ref, pts9_ref, dv_ref, dx_ref, dn_ref, o_ref, *, V, n, kc):
    idx = idx_ref[0]
    pts9 = pts9_ref[0]
    dirs = [_norm_cols(dv_ref[...]), _norm_cols(dx_ref[...]), _norm_cols(dn_ref[...])]
    lane = jax.lax.broadcasted_iota(jnp.int32, (V, V), 1)
    pts9_parts = _split3(pts9)
    accs = [None, None, None]
    for j in range(n):
        oh = _onehot(idx[:, j], lane)
        nb9 = _gather3(oh, pts9_parts)
        for br in range(3):
            pts = pts9[:, 3 * br:3 * br + 3]
            d = nb9[:, 3 * br:3 * br + 3] - pts
            nn = jnp.maximum(jnp.sqrt(jnp.sum(d * d, axis=1, keepdims=True)), 1e-12)
            th = jnp.maximum(_mmul(d / nn, dirs[br]), 0.0)
            accs[br] = th if j == 0 else jnp.maximum(accs[br], th)
    for br in range(3):
        o_ref[0, :, br * kc:(br + 1) * kc] = jnp.maximum(_sup_sum(accs[br], kc), 0.0)


def _conv_surface3(idx, pts9, d0v, d0x, d0n, kc):
    B, V, _ = pts9.shape
    n = idx.shape[2]
    return pl.pallas_call(
        functools.partial(_surface_kernel, V=V, n=n, kc=kc),
        grid=(B,),
        in_specs=[
            pl.BlockSpec((1, V, n), lambda b: (b, 0, 0)),
            pl.BlockSpec((1, V, 9), lambda b: (b, 0, 0)),
            pl.BlockSpec(d0v.shape, lambda b: (0, 0)),
            pl.BlockSpec(d0x.shape, lambda b: (0, 0)),
            pl.BlockSpec(d0n.shape, lambda b: (0, 0)),
        ],
        out_specs=pl.BlockSpec((1, V, 3 * kc), lambda b: (b, 0, 0)),
        out_shape=jax.ShapeDtypeStruct((B, V, 3 * kc), jnp.float32),
    )(idx, pts9, d0v, d0x, d0n)


def _conv3_kernel(idx_ref, pts9_ref, fm_ref, wv_ref, bv_ref, dv_ref,
                  wx_ref, bx_ref, dx_ref, wn_ref, bn_ref, dn_ref, o_ref,
                  *, V, VB, n, inc, outc):
    g = pl.program_id(1)
    idx = idx_ref[0]
    pts9 = pts9_ref[0]
    pts9_blk = pts9_ref[0, pl.ds(g * VB, VB), :]
    wbd = [(wv_ref, bv_ref, dv_ref), (wx_ref, bx_ref, dx_ref),
           (wn_ref, bn_ref, dn_ref)]
    centers, sup_parts, dirs = [], [], []
    for br, (w_r, b_r, d_r) in enumerate(wbd):
        fm_br = fm_ref[0][:, br * inc:(br + 1) * inc]
        fm_blk = fm_ref[0, pl.ds(g * VB, VB), br * inc:(br + 1) * inc]
        centers.append(_mmul(fm_blk, w_r[...][:, :outc]) + b_r[...][:, :outc])
        sup_parts.append(_split2(_mmul(fm_br, w_r[...][:, outc:])
                                 + b_r[...][:, outc:]))
        dirs.append(_norm_cols(d_r[...]))
    pts9_parts = _split3(pts9)
    sc = (SUP + 1) * outc - outc
    table = jnp.concatenate([sup_parts[0][0], sup_parts[0][1],
                             sup_parts[1][0], sup_parts[1][1],
                             sup_parts[2][0], sup_parts[2][1],
                             pts9_parts[0], pts9_parts[1], pts9_parts[2]],
                            axis=1)
    lane = jax.lax.broadcasted_iota(jnp.int32, (VB, V), 1)
    accs = [None, None, None]
    for j in range(n):
        oh = _onehot(idx[:, j], lane)
        G = jnp.dot(oh, table, preferred_element_type=jnp.float32)
        p0 = 6 * sc
        nb9 = G[:, p0:p0 + 9] + G[:, p0 + 9:p0 + 18] + G[:, p0 + 18:p0 + 27]
        for br in range(3):
            pts = pts9_blk[:, 3 * br:3 * br + 3]
            d = nb9[:, 3 * br:3 * br + 3] - pts
            nn = jnp.maximum(jnp.sqrt(jnp.sum(d * d, axis=1, keepdims=True)), 1e-12)
            th = jnp.maximum(_mmul(d / nn, dirs[br]), 0.0)
            t = th * (G[:, 2 * br * sc:(2 * br + 1) * sc]
                      + G[:, (2 * br + 1) * sc:(2 * br + 2) * sc])
            accs[br] = t if j == 0 else jnp.maximum(accs[br], t)
    for br in range(3):
        o_ref[0, :, br * outc:(br + 1) * outc] = centers[br] + _sup_sum(accs[br], outc)


def _conv3(idx, pts9, fm3, wv, bv, dv, wx, bx, dx, wn, bn, dn, outc, blocks=1):
    B, V, _ = pts9.shape
    n = idx.shape[2]
    inc = fm3.shape[2] // 3
    VB = V // blocks
    bvr, bxr, bnr = bv.reshape(1, -1), bx.reshape(1, -1), bn.reshape(1, -1)
    full = lambda a: pl.BlockSpec(a.shape, lambda bb, gg: tuple(0 for _ in a.shape))
    return pl.pallas_call(
        functools.partial(_conv3_kernel, V=V, VB=VB, n=n, inc=inc, outc=outc),
        grid=(B, blocks),
        in_specs=[
            pl.BlockSpec((1, VB, n), lambda bb, gg: (bb, gg, 0)),
            pl.BlockSpec((1, V, 9), lambda bb, gg: (bb, 0, 0)),
            pl.BlockSpec((1, V, 3 * inc), lambda bb, gg: (bb, 0, 0)),
            full(wv), full(bvr), full(dv),
            full(wx), full(bxr), full(dx),
            full(wn), full(bnr), full(dn),
        ],
        out_specs=pl.BlockSpec((1, VB, 3 * outc), lambda bb, gg: (bb, gg, 0)),
        out_shape=jax.ShapeDtypeStruct((B, V, 3 * outc), jnp.float32),
    )(idx, pts9, fm3, wv, bvr, dv, wx, bxr, dx, wn, bnr, dn)


def _conv_layer_kernel(idx_ref, pts_ref, fmap_ref, w_ref, b_ref, d_ref, o_ref, *, V, n, outc):
    idx = idx_ref[0]
    pts = pts_ref[0]
    dirs = _norm_cols(d_ref[...])
    fo = _mmul(fmap_ref[0], w_ref[...]) + b_ref[...]
    center = fo[:, :outc]
    sup_parts = _split3(fo[:, outc:])
    pts_parts = _split3(pts)
    lane = jax.lax.broadcasted_iota(jnp.int32, (V, V), 1)
    acc = None
    for j in range(n):
        oh = _onehot(idx[:, j], lane)
        th = _nd_theta(oh, pts_parts, pts, dirs)
        g = _gather3(oh, sup_parts)
        t = th * g
        acc = t if j == 0 else jnp.maximum(acc, t)
    o_ref[0] = center + _sup_sum(acc, outc)


def _conv_layer(idx, pts, fmap, w, b, d, outc):
    B, V, D = pts.shape
    n = idx.shape[2]
    inc = fmap.shape[2]
    b2 = b.reshape(1, -1)
    return pl.pallas_call(
        functools.partial(_conv_layer_kernel, V=V, n=n, outc=outc),
        grid=(B,),
        in_specs=[
            pl.BlockSpec((1, V, n), lambda bb: (bb, 0, 0)),
            pl.BlockSpec((1, V, D), lambda bb: (bb, 0, 0)),
            pl.BlockSpec((1, V, inc), lambda bb: (bb, 0, 0)),
            pl.BlockSpec(w.shape, lambda bb: (0, 0)),
            pl.BlockSpec(b2.shape, lambda bb: (0, 0)),
            pl.BlockSpec(d.shape, lambda bb: (0, 0)),
        ],
        out_specs=pl.BlockSpec((1, V, outc), lambda bb: (bb, 0, 0)),
        out_shape=jax.ShapeDtypeStruct((B, V, outc), jnp.float32),
    )(idx, pts, fmap, w, b2, d)


def _bn_relu_kernel(x_ref, o_ref):
    x = x_ref[...]
    m = jnp.mean(x, axis=0, keepdims=True)
    v = jnp.mean((x - m) * (x - m), axis=0, keepdims=True)
    o_ref[...] = jnp.maximum((x - m) / jnp.sqrt(v + 1e-5), 0.0)


def _bn_relu(x):
    B, V, C = x.shape
    x2 = x.reshape(B * V, C)
    y = pl.pallas_call(
        _bn_relu_kernel,
        in_specs=[pl.BlockSpec(x2.shape, lambda: (0, 0))],
        out_specs=pl.BlockSpec(x2.shape, lambda: (0, 0)),
        out_shape=jax.ShapeDtypeStruct(x2.shape, jnp.float32),
    )(x2)
    return y.reshape(B, V, C)


def _pool_kernel(ptsT_ref, rows_ref, fmap_ref, o_ref, *, V, R, nnum):
    ptsT = ptsT_ref[0]
    rows = rows_ref[0]
    fmap = fmap_ref[0]
    dist = _dist(rows, ptsT)
    fmap_parts = _split2(fmap)
    lane = jax.lax.broadcasted_iota(jnp.int32, (R, V), 1)
    acc = None
    for j in range(nnum + 1):
        amin = _argmin_cols(dist, lane, V)
        if j >= 1:
            oh = _onehot(amin, lane)
            g = _gather3(oh, fmap_parts)
            acc = g if j == 1 else jnp.maximum(acc, g)
        dist = jnp.where(lane == amin[:, None], BIG, dist)
    o_ref[0] = acc


def _pool(pts, fmap, rate=4, nnum=4):
    B, V, D = pts.shape
    C = fmap.shape[2]
    R = V // rate
    rows = pts[:, ::rate, :]
    ptsT = jnp.transpose(pts, (0, 2, 1))
    return pl.pallas_call(
        functools.partial(_pool_kernel, V=V, R=R, nnum=nnum),
        grid=(B,),
        in_specs=[
            pl.BlockSpec((1, D, V), lambda b: (b, 0, 0)),
            pl.BlockSpec((1, R, D), lambda b: (b, 0, 0)),
            pl.BlockSpec((1, V, C), lambda b: (b, 0, 0)),
        ],
        out_specs=pl.BlockSpec((1, R, C), lambda b: (b, 0, 0)),
        out_shape=jax.ShapeDtypeStruct((B, R, C), jnp.float32),
    )(ptsT, rows, fmap)


def _up_kernel(tgt_ref, srcT_ref, feat_ref, o_ref, *, R):
    tgt = tgt_ref[0]
    srcT = srcT_ref[0]
    T = tgt.shape[0]
    dist = _dist(tgt, srcT)
    lane = jax.lax.broadcasted_iota(jnp.int32, (T, R), 1)
    amin = _argmin_cols(dist, lane, R)
    oh = _onehot(amin, lane)
    o_ref[0] = _gather3(oh, _split2(feat_ref[0]))


def _upsample(tgt, src, feat):
    B, T, D = tgt.shape
    R, C = src.shape[1], feat.shape[2]
    srcT = jnp.transpose(src, (0, 2, 1))
    return pl.pallas_call(
        functools.partial(_up_kernel, R=R),
        grid=(B,),
        in_specs=[
            pl.BlockSpec((1, T, D), lambda b: (b, 0, 0)),
            pl.BlockSpec((1, D, R), lambda b: (b, 0, 0)),
            pl.BlockSpec((1, R, C), lambda b: (b, 0, 0)),
        ],
        out_specs=pl.BlockSpec((1, T, C), lambda b: (b, 0, 0)),
        out_shape=jax.ShapeDtypeStruct((B, T, C), jnp.float32),
    )(tgt, srcT, feat)


def kernel(vertices, xyz, normal, d0v, d0x, d0n, w1v, b1v, d1v, w1x, b1x, d1x,
           w1n, b1n, d1n, w2v, b2v, d2v, w2x, b2x, d2x, w2n, b2n, d2n,
           w3v, b3v, d3v, w3x, b3x, d3x, w3n, b3n, d3n, w4, b4, d4, w5, b5, d5):
    B, N, _ = vertices.shape
    pts9 = jnp.concatenate([vertices, xyz, normal], axis=2)

    idx = _knn(vertices, NEI)
    fm0 = _conv_surface3(idx, pts9, d0v, d0x, d0n, 128)
    fm0v, fm0x, fm0n = fm0[:, :, :128], fm0[:, :, 128:256], fm0[:, :, 256:]

    feat_1 = _bn_relu(_conv3(idx, pts9, fm0, w1v, b1v, d1v, w1x, b1x, d1x,
                             w1n, b1n, d1n, 128, blocks=2))
    fm1v, fm1x, fm1n = (feat_1[:, :, :128], feat_1[:, :, 128:256],
                        feat_1[:, :, 256:])

    v_pool_1 = vertices[:, ::4, :]
    x_pool_1 = xyz[:, ::4, :]
    n_pool_1 = normal[:, ::4, :]
    pool_1 = pts9[:, ::4, :]
    fmp_v = _pool(vertices, fm1v)
    fmp_x = _pool(xyz, fm1x)
    fmp_n = _pool(normal, fm1n)

    k2 = min(NEI, v_pool_1.shape[1] // 8)
    idx2 = _knn(v_pool_1, k2)
    pts9p = pts9[:, ::4, :]
    fmp = jnp.concatenate([fmp_v, fmp_x, fmp_n], axis=2)
    fm2 = _bn_relu(_conv3(idx2, pts9p, fmp, w2v, b2v, d2v, w2x, b2x, d2x,
                          w2n, b2n, d2n, 256))
    feat_2 = _bn_relu(_conv3(idx2, pts9p, fm2, w3v, b3v, d3v, w3x, b3x, d3x,
                             w3n, b3n, d3n, 256))

    pool_2 = pool_1[:, ::4, :]
    fm_pool_2 = _pool(pool_1, feat_2)

    k3 = min(NEI, pool_2.shape[1] // 8)
    idx3 = _knn(pool_2, k3)
    fm4 = _conv_layer(idx3, pool_2, fm_pool_2, w4, b4, d4, 256)
    fm5 = _conv_layer(idx3, pool_2, fm4, w5, b5, d5, 512)

    feat_12 = jnp.concatenate([feat_1[:, :feat_2.shape[1], :], feat_2], axis=2)
    f12 = _upsample(vertices, pool_1[:, :, :3], feat_12)
    f5 = _upsample(vertices, pool_2[:, :, :3], fm5)
    return jnp.concatenate([f5, f12], axis=2)
